# baseline (speedup 1.0000x reference)
import functools

import jax
import jax.numpy as jnp
from jax import lax
from jax.experimental import pallas as pl
from jax.experimental.pallas import tpu as pltpu
from jax.experimental.pallas import tpu_sc as plsc

_D = 1024
_CH = 32
_LANES = 16


def _sc_body(x_hbm, pos_hbm, pe_hbm, out_hbm, idx_v, x_v, pe_v, sem):
    nc = 2
    rows_w = 256
    c = lax.axis_index("c")
    s = lax.axis_index("s")
    wid = s * nc + c
    base = wid * rows_w
    soff = lax.rem(base, 2048)

    def chunk(ci, _):
        rb = base + ci * _CH
        sb = soff + ci * _CH
        pltpu.sync_copy(pos_hbm.at[pl.ds(sb, _CH)], idx_v)
        pltpu.sync_copy(x_hbm.at[pl.ds(rb, _CH)], x_v)
        pltpu.async_copy(pe_hbm.at[idx_v], pe_v, sem).wait()

        def add_cols(j, _):
            col = j * _LANES
            for r in range(_CH):
                sl = pl.ds(col, _LANES)
                x_v[r, sl] = x_v[r, sl] + pe_v[r, sl]
            return 0

        lax.fori_loop(0, _D // _LANES, add_cols, 0)
        pltpu.sync_copy(x_v, out_hbm.at[pl.ds(rb, _CH)])
        return 0

    lax.fori_loop(0, rows_w // _CH, chunk, 0)


def kernel(x, pe_table, pos_arange):
    B, S, D = x.shape
    x2 = x.reshape(B * S, D)
    pos = pos_arange.astype(jnp.int32).reshape(-1)
    mesh = plsc.VectorSubcoreMesh(core_axis_name="c", subcore_axis_name="s")
    k = functools.partial(
        pl.kernel,
        mesh=mesh,
        out_type=jax.ShapeDtypeStruct((B * S, D), jnp.float32),
        scratch_types=[
            pltpu.VMEM((_CH,), jnp.int32),
            pltpu.VMEM((_CH, D), jnp.float32),
            pltpu.VMEM((_CH, D), jnp.float32),
            pltpu.SemaphoreType.DMA,
        ],
    )(_sc_body)
    out = k(x2, pos, pe_table)
    return out.reshape(B, S, D)


# speedup vs baseline: 1.5322x; 1.5322x over previous
import functools

import jax
import jax.numpy as jnp
from jax import lax
from jax.experimental import pallas as pl
from jax.experimental.pallas import tpu as pltpu
from jax.experimental.pallas import tpu_sc as plsc

_D = 1024
_CH = 16
_NBUF = 2
_LANES = 16
_ROWS_W = 256


def _sc_body(x_hbm, pos_hbm, pe_hbm, out_hbm,
             idx_v, x_v0, x_v1, pe_v0, pe_v1,
             lsem0, lsem1, gsem0, gsem1, osem0, osem1):
    nc = 2
    c = lax.axis_index("c")
    s = lax.axis_index("s")
    wid = s * nc + c
    base = wid * _ROWS_W
    soff = lax.rem(base, 2048)
    nchunks = _ROWS_W // _CH

    x_bufs = (x_v0, x_v1)
    pe_bufs = (pe_v0, pe_v1)
    lsems = (lsem0, lsem1)
    gsems = (gsem0, gsem1)
    osems = (osem0, osem1)

    pltpu.sync_copy(pos_hbm.at[pl.ds(soff, _ROWS_W)], idx_v)

    def start_loads(ci, b):
        rb = base + ci * _CH
        pltpu.async_copy(x_hbm.at[pl.ds(rb, _CH)], x_bufs[b], lsems[b])
        pltpu.async_copy(pe_hbm.at[idx_v.at[pl.ds(ci * _CH, _CH)]],
                         pe_bufs[b], gsems[b])

    for b in range(_NBUF):
        start_loads(b, b)

    def outer(g, _):
        for b in range(_NBUF):
            ci = g * _NBUF + b
            xb, pb = x_bufs[b], pe_bufs[b]
            pltpu.make_async_copy(x_hbm.at[pl.ds(0, _CH)], xb, lsems[b]).wait()
            pltpu.make_async_copy(pe_hbm.at[idx_v.at[pl.ds(0, _CH)]],
                                  pb, gsems[b]).wait()

            def add_cols(j, _):
                col = j * _LANES
                for r in range(_CH):
                    sl = pl.ds(col, _LANES)
                    xb[r, sl] = xb[r, sl] + pb[r, sl]
                return 0

            lax.fori_loop(0, _D // _LANES, add_cols, 0)

            rb = base + ci * _CH
            pltpu.async_copy(xb, out_hbm.at[pl.ds(rb, _CH)], osems[b])

            @pl.when(ci + _NBUF < nchunks)
            def _():
                pltpu.make_async_copy(xb, out_hbm.at[pl.ds(rb, _CH)],
                                      osems[b]).wait()
                start_loads(ci + _NBUF, b)
        return 0

    lax.fori_loop(0, nchunks // _NBUF, outer, 0)

    for b in range(_NBUF):
        pltpu.make_async_copy(x_bufs[b], out_hbm.at[pl.ds(0, _CH)],
                              osems[b]).wait()


def kernel(x, pe_table, pos_arange):
    B, S, D = x.shape
    x2 = x.reshape(B * S, D)
    pos = pos_arange.astype(jnp.int32).reshape(-1)
    mesh = plsc.VectorSubcoreMesh(core_axis_name="c", subcore_axis_name="s")
    k = functools.partial(
        pl.kernel,
        mesh=mesh,
        out_type=jax.ShapeDtypeStruct((B * S, D), jnp.float32),
        scratch_types=[
            pltpu.VMEM((_ROWS_W,), jnp.int32),
            pltpu.VMEM((_CH, D), jnp.float32),
            pltpu.VMEM((_CH, D), jnp.float32),
            pltpu.VMEM((_CH, D), jnp.float32),
            pltpu.VMEM((_CH, D), jnp.float32),
            pltpu.SemaphoreType.DMA,
            pltpu.SemaphoreType.DMA,
            pltpu.SemaphoreType.DMA,
            pltpu.SemaphoreType.DMA,
            pltpu.SemaphoreType.DMA,
            pltpu.SemaphoreType.DMA,
        ],
    )(_sc_body)
    out = k(x2, pos, pe_table)
    return out.reshape(B, S, D)


# speedup vs baseline: 1.5632x; 1.0202x over previous
import functools

import jax
import jax.numpy as jnp
from jax import lax
from jax.experimental import pallas as pl
from jax.experimental.pallas import tpu as pltpu
from jax.experimental.pallas import tpu_sc as plsc

_D = 1024
_CH = 16
_NBUF = 3
_LANES = 16
_ROWS_W = 256
_NCHUNKS = _ROWS_W // _CH


def _sc_body(x_hbm, pos_hbm, pe_hbm, out_hbm, idx_v, *bufs_and_sems):
    x_bufs = bufs_and_sems[0:3]
    pe_bufs = bufs_and_sems[3:6]
    lsems = bufs_and_sems[6:9]
    gsems = bufs_and_sems[9:12]
    osems = bufs_and_sems[12:15]

    nc = 2
    c = lax.axis_index("c")
    s = lax.axis_index("s")
    wid = s * nc + c
    base = wid * _ROWS_W
    soff = lax.rem(base, 2048)

    pltpu.sync_copy(pos_hbm.at[pl.ds(soff, _ROWS_W)], idx_v)

    def start_loads(ci, b):
        rb = base + ci * _CH
        pltpu.async_copy(x_hbm.at[pl.ds(rb, _CH)], x_bufs[b], lsems[b])
        pltpu.async_copy(pe_hbm.at[idx_v.at[pl.ds(ci * _CH, _CH)]],
                         pe_bufs[b], gsems[b])

    for b in range(_NBUF):
        start_loads(b, b)

    for ci in range(_NCHUNKS):
        b = ci % _NBUF
        xb, pb = x_bufs[b], pe_bufs[b]
        rb = base + ci * _CH
        pltpu.make_async_copy(x_hbm.at[pl.ds(rb, _CH)], xb, lsems[b]).wait()
        pltpu.make_async_copy(pe_hbm.at[idx_v.at[pl.ds(0, _CH)]],
                              pb, gsems[b]).wait()

        def add_cols(j, _, xb=xb, pb=pb):
            col = j * _LANES
            for r in range(_CH):
                sl = pl.ds(col, _LANES)
                xb[r, sl] = xb[r, sl] + pb[r, sl]
            return 0

        lax.fori_loop(0, _D // _LANES, add_cols, 0)
        pltpu.async_copy(xb, out_hbm.at[pl.ds(rb, _CH)], osems[b])

        if ci >= 1 and ci + _NBUF - 1 < _NCHUNKS:
            pv = (ci - 1) % _NBUF
            prb = base + (ci - 1) * _CH
            pltpu.make_async_copy(x_bufs[pv], out_hbm.at[pl.ds(prb, _CH)],
                                  osems[pv]).wait()
            start_loads(ci + _NBUF - 1, pv)

    for ci in range(_NCHUNKS - _NBUF, _NCHUNKS):
        b = ci % _NBUF
        rb = base + ci * _CH
        pltpu.make_async_copy(x_bufs[b], out_hbm.at[pl.ds(rb, _CH)],
                              osems[b]).wait()


def kernel(x, pe_table, pos_arange):
    B, S, D = x.shape
    x2 = x.reshape(B * S, D)
    pos = pos_arange.astype(jnp.int32).reshape(-1)
    mesh = plsc.VectorSubcoreMesh(core_axis_name="c", subcore_axis_name="s")
    k = functools.partial(
        pl.kernel,
        mesh=mesh,
        out_type=jax.ShapeDtypeStruct((B * S, D), jnp.float32),
        scratch_types=(
            [pltpu.VMEM((_ROWS_W,), jnp.int32)]
            + [pltpu.VMEM((_CH, D), jnp.float32)] * 6
            + [pltpu.SemaphoreType.DMA] * 9
        ),
    )(_sc_body)
    out = k(x2, pos, pe_table)
    return out.reshape(B, S, D)


# speedup vs baseline: 3.9886x; 2.5516x over previous
import jax
import jax.numpy as jnp
from jax.experimental import pallas as pl
from jax.experimental.pallas import tpu as pltpu

_BS = 512


def _pe_add_kernel(pos_ref, x_ref, pe_ref, o_ref):
    o_ref[...] = x_ref[...] + pe_ref[...][None, :, :]


def kernel(x, pe_table, pos_arange):
    B, S, D = x.shape
    nblk = S // _BS
    pos = pos_arange.astype(jnp.int32).reshape(-1)

    grid_spec = pltpu.PrefetchScalarGridSpec(
        num_scalar_prefetch=1,
        grid=(nblk,),
        in_specs=[
            pl.BlockSpec((B, _BS, D), lambda j, pos_ref: (0, j, 0)),
            pl.BlockSpec((_BS, D), lambda j, pos_ref: (pos_ref[j * _BS] // _BS, 0)),
        ],
        out_specs=pl.BlockSpec((B, _BS, D), lambda j, pos_ref: (0, j, 0)),
    )
    return pl.pallas_call(
        _pe_add_kernel,
        grid_spec=grid_spec,
        out_shape=jax.ShapeDtypeStruct(x.shape, x.dtype),
        compiler_params=pltpu.CompilerParams(
            dimension_semantics=("parallel",),
        ),
    )(pos, x, pe_table)
